# baseline (speedup 1.0000x reference)
import functools

import jax
import jax.numpy as jnp
from jax import lax
from jax.experimental import pallas as pl
from jax.experimental.pallas import tpu as pltpu
from jax.experimental.pallas import tpu_sc as plsc

TOPK = 8
NUM_EXPERTS = 64
ROW_BLOCK = 1024
N_ROWS = 32768
N_WORKERS = 32
GROUP = 16
CHUNKS = 4
CHUNK_ROWS = N_ROWS // CHUNKS
ROWS_PER_W = CHUNK_ROWS // N_WORKERS
STAGE = ROWS_PER_W


def _mm_block(x_ref, w_ref, b_ref, s_ref):
    s_ref[...] = jnp.dot(x_ref[...], w_ref[...],
                         preferred_element_type=jnp.float32) + b_ref[...]


def _tc_scores(inputs, W, b):
    n_rows = inputs.shape[0]
    return pl.pallas_call(
        _mm_block,
        grid=(n_rows // ROW_BLOCK,),
        in_specs=[
            pl.BlockSpec((ROW_BLOCK, inputs.shape[1]), lambda i: (i, 0)),
            pl.BlockSpec((inputs.shape[1], NUM_EXPERTS), lambda i: (0, 0)),
            pl.BlockSpec((1, NUM_EXPERTS), lambda i: (0, 0)),
        ],
        out_specs=pl.BlockSpec((ROW_BLOCK, NUM_EXPERTS), lambda i: (i, 0)),
        out_shape=jax.ShapeDtypeStruct((n_rows, NUM_EXPERTS), jnp.float32),
    )(inputs, W, b.reshape(1, NUM_EXPERTS))


def _sc_body(scores_hbm, probs_hbm, idx_hbm, sbuf, pbuf, ibuf):
    wid = lax.axis_index("s") * 2 + lax.axis_index("c")
    base = wid * ROWS_PER_W
    lane = lax.broadcasted_iota(jnp.int32, (GROUP,), 0)
    neg_inf = jnp.full((GROUP,), -jnp.inf, jnp.float32)
    zeros_i = jnp.zeros((GROUP,), jnp.int32)

    def insert(v, e, t, x):
        col_idx = jnp.full((GROUP,), e, jnp.int32)
        c = [v > t[j] for j in range(TOPK)]
        nt, nx = [], []
        for j in range(TOPK):
            if j == 0:
                cand_t, cand_x = v, col_idx
            else:
                cand_t = jnp.where(c[j - 1], t[j - 1], v)
                cand_x = jnp.where(c[j - 1], x[j - 1], col_idx)
            nt.append(jnp.where(c[j], cand_t, t[j]))
            nx.append(jnp.where(c[j], cand_x, x[j]))
        return nt, nx

    row0 = base
    pltpu.sync_copy(
        scores_hbm.at[pl.ds(row0 * NUM_EXPERTS, STAGE * NUM_EXPERTS)], sbuf)

    def sub_body(sub, _):
        rows = sub * GROUP + lane
        gbase = rows * NUM_EXPERTS
        t = [neg_inf] * TOPK
        x = [zeros_i] * TOPK
        for e in range(NUM_EXPERTS):
            v = plsc.load_gather(sbuf, [gbase + e])
            t, x = insert(v, e, t, x)
        es = [jnp.exp(t[j] - t[0]) for j in range(TOPK)]
        ssum = es[0]
        for j in range(1, TOPK):
            ssum = ssum + es[j]
        obase = rows * TOPK
        for j in range(TOPK):
            plsc.store_scatter(pbuf, [obase + j], es[j] / ssum)
            plsc.store_scatter(ibuf, [obase + j], x[j])
        return 0

    lax.fori_loop(0, STAGE // GROUP, sub_body, 0)
    pltpu.sync_copy(pbuf, probs_hbm.at[pl.ds(row0 * TOPK, STAGE * TOPK)])
    pltpu.sync_copy(ibuf, idx_hbm.at[pl.ds(row0 * TOPK, STAGE * TOPK)])


_sc_router = functools.partial(
    pl.kernel,
    mesh=plsc.VectorSubcoreMesh(core_axis_name="c", subcore_axis_name="s"),
    out_type=[
        jax.ShapeDtypeStruct((CHUNK_ROWS * TOPK,), jnp.float32),
        jax.ShapeDtypeStruct((CHUNK_ROWS * TOPK,), jnp.int32),
    ],
    scratch_types=[
        pltpu.VMEM((STAGE * NUM_EXPERTS,), jnp.float32),
        pltpu.VMEM((STAGE * TOPK,), jnp.float32),
        pltpu.VMEM((STAGE * TOPK,), jnp.int32),
    ],
    compiler_params=pltpu.CompilerParams(needs_layout_passes=False),
)(_sc_body)


@jax.jit
def kernel(inputs, W, b):
    probs = []
    idx = []
    for c in range(CHUNKS):
        xc = lax.slice_in_dim(inputs, c * CHUNK_ROWS, (c + 1) * CHUNK_ROWS)
        scores_c = _tc_scores(xc, W, b)
        p_c, i_c = _sc_router(scores_c.reshape(-1))
        probs.append(p_c.reshape(CHUNK_ROWS, TOPK))
        idx.append(i_c.reshape(CHUNK_ROWS, TOPK))
    return (jnp.concatenate(probs, axis=0), jnp.concatenate(idx, axis=0))


# speedup vs baseline: 1.9013x; 1.9013x over previous
import functools

import jax
import jax.numpy as jnp
from jax.experimental import pallas as pl

TOPK = 8
NUM_EXPERTS = 64
ROW_BLOCK = 2048


def _router_block(x_ref, w_ref, b_ref, probs_ref, idx_ref):
    x = x_ref[...]
    w = w_ref[...]
    scores = jnp.dot(x, w, preferred_element_type=jnp.float32) + b_ref[...]

    rows = scores.shape[0]
    iota = jax.lax.broadcasted_iota(jnp.int32, (rows, NUM_EXPERTS), 1).astype(
        jnp.float32)
    vals = scores
    top_vals = []
    top_idx = []
    for k in range(TOPK):
        m = jnp.max(vals, axis=1, keepdims=True)
        idx = jnp.min(jnp.where(vals == m, iota, float(NUM_EXPERTS)), axis=1,
                      keepdims=True)
        top_vals.append(m)
        top_idx.append(idx)
        if k + 1 < TOPK:
            vals = jnp.where(iota == idx, -jnp.inf, vals)

    lane8 = jax.lax.broadcasted_iota(jnp.int32, (rows, TOPK), 1)
    v = top_vals[0]
    ix = top_idx[0]
    for k in range(1, TOPK):
        sel = lane8 == k
        v = jnp.where(sel, top_vals[k], v)
        ix = jnp.where(sel, top_idx[k], ix)
    e = jnp.exp(v - top_vals[0])
    probs_ref[...] = e / jnp.sum(e, axis=1, keepdims=True)
    idx_ref[...] = ix.astype(jnp.int32)


@jax.jit
def kernel(inputs, W, b):
    n_rows = inputs.shape[0]
    grid = (n_rows // ROW_BLOCK,)
    probs, idx = pl.pallas_call(
        _router_block,
        grid=grid,
        in_specs=[
            pl.BlockSpec((ROW_BLOCK, inputs.shape[1]), lambda i: (i, 0)),
            pl.BlockSpec((inputs.shape[1], NUM_EXPERTS), lambda i: (0, 0)),
            pl.BlockSpec((1, NUM_EXPERTS), lambda i: (0, 0)),
        ],
        out_specs=[
            pl.BlockSpec((ROW_BLOCK, TOPK), lambda i: (i, 0)),
            pl.BlockSpec((ROW_BLOCK, TOPK), lambda i: (i, 0)),
        ],
        out_shape=[
            jax.ShapeDtypeStruct((n_rows, TOPK), jnp.float32),
            jax.ShapeDtypeStruct((n_rows, TOPK), jnp.int32),
        ],
    )(inputs, W, b.reshape(1, NUM_EXPERTS))
    return probs, idx
